# Initial kernel scaffold; baseline (speedup 1.0000x reference)
import jax
import jax.numpy as jnp
from jax.experimental import pallas as pl


def kernel(x, pos, edge_index, W1, b1, rm1, rv1, g1, beta1, W2, b2, rm2, rv2, g2, beta2):
    raise NotImplementedError("write your pallas kernel here")


# speedup vs baseline: 9.9058x; 9.9058x over previous
import jax
import jax.numpy as jnp
from jax import lax
from jax.experimental import pallas as pl
from jax.experimental.pallas import tpu as pltpu
from jax.experimental.pallas import tpu_sc as plsc

N = 50000
E = 800000
CELL_INV = 1.0 / 16.0
GRID_W = 8
NUM_GRIDS = 64
EPS = 1e-5

NPAD = 51200
RPT = NPAD // 16
EPAD = 802816
NBW = EPAD // 16 // 128
NQ = 4
NBQ = NBW // NQ
BA = 2048
NBLK = NPAD // BA
FH = 16


def _make_seg_sum():
    mesh = plsc.VectorSubcoreMesh(core_axis_name="c", subcore_axis_name="s")

    def body(tab_hbm, srcg_hbm, dstg_hbm, dsts_hbm, z_hbm, s_out,
             srcv, dstgv, dstsv, rowsu, rowsw, semu, semw, acc):
        c = lax.axis_index("c")
        t = lax.axis_index("s")
        pltpu.sync_copy(z_hbm, acc.at[pl.ds(t * RPT, RPT)])
        plsc.subcore_barrier()

        def chunk(q, carry0):
            pltpu.sync_copy(srcg_hbm.at[c, t, pl.ds(q * NBQ, NBQ)], srcv)
            pltpu.sync_copy(dstg_hbm.at[c, t, pl.ds(q * NBQ, NBQ)], dstgv)
            pltpu.sync_copy(dsts_hbm.at[t, pl.ds(q * NBQ, NBQ)], dstsv)
            pltpu.async_copy(tab_hbm.at[srcv.at[0]], rowsu.at[0], semu)
            pltpu.async_copy(tab_hbm.at[dstgv.at[0]], rowsw.at[0], semw)

            def step(j, carry):
                pltpu.make_async_copy(
                    tab_hbm.at[srcv.at[j]], rowsu.at[j % 2], semu).wait()
                pltpu.make_async_copy(
                    tab_hbm.at[dstgv.at[j]], rowsw.at[j % 2], semw).wait()

                @pl.when(j + 1 < NBQ)
                def _():
                    pltpu.async_copy(
                        tab_hbm.at[srcv.at[j + 1]], rowsu.at[(j + 1) % 2],
                        semu)
                    pltpu.async_copy(
                        tab_hbm.at[dstgv.at[j + 1]], rowsw.at[(j + 1) % 2],
                        semw)

                pltpu.sync_copy(rowsu.at[j % 2], acc.at[dstsv.at[j]],
                                add=True)
                pltpu.sync_copy(rowsw.at[j % 2], acc.at[dstsv.at[j]],
                                add=True)
                return carry

            lax.fori_loop(0, NBQ, step, 0)
            return carry0

        lax.fori_loop(0, NQ, chunk, 0)
        plsc.subcore_barrier()
        pltpu.sync_copy(acc.at[pl.ds(t * RPT, RPT)],
                        s_out.at[pl.ds(c * NPAD + t * RPT, RPT)])

    return pl.kernel(
        body,
        out_type=(jax.ShapeDtypeStruct((2 * NPAD, FH), jnp.float32),),
        scratch_types=[
            pltpu.VMEM((NBQ, 128), jnp.int32),
            pltpu.VMEM((NBQ, 128), jnp.int32),
            pltpu.VMEM((NBQ, 128), jnp.int32),
            pltpu.VMEM((2, 128, FH), jnp.float32),
            pltpu.VMEM((2, 128, FH), jnp.float32),
            pltpu.SemaphoreType.DMA,
            pltpu.SemaphoreType.DMA,
            pltpu.VMEM_SHARED((NPAD, FH), jnp.float32),
        ],
        mesh=mesh,
        compiler_params=pltpu.CompilerParams(use_tc_tiling_on_sc=False))


def _prep_body(x_ref, pos_ref, w1_ref, w2p_ref, b1_ref, b2_ref,
               t1_ref, wn2_ref, cell_ref):
    xv = x_ref[...]
    pv = pos_ref[...]
    w1 = w1_ref[...]
    w2p = w2p_ref[...]
    px = pv[:, 0:1]
    py = pv[:, 1:2]
    p1 = px * w1[1:2, :] + py * w1[2:3, :]
    u1 = xv * w1[0:1, :] + p1
    wn1 = b1_ref[...] - p1
    t1_ref[...] = jnp.stack([u1, u1, wn1, wn1], axis=0)
    wn2_ref[...] = b2_ref[...] - (px * w2p[0:1, :] + py * w2p[1:2, :])
    ix = jnp.clip(jnp.floor(px * CELL_INV), 0, GRID_W - 1).astype(jnp.int32)
    iy = jnp.clip(jnp.floor(py * CELL_INV), 0, GRID_W - 1).astype(jnp.int32)
    cell_ref[...] = iy * GRID_W + ix


def _mid_body(s_ref, wn2_ref, w2h_ref, b2_ref,
              rm1_ref, rv1_ref, g1_ref, be1_ref, t2_ref):
    s = s_ref[...][0]
    scale = g1_ref[...] * lax.rsqrt(rv1_ref[...] + EPS)
    h1 = jnp.maximum((s - rm1_ref[...]) * scale + be1_ref[...], 0.0)
    wn2 = wn2_ref[...]
    u2 = jnp.dot(h1, w2h_ref[...],
                 preferred_element_type=jnp.float32) + (b2_ref[...] - wn2)
    t2_ref[...] = jnp.stack([u2[:, :16], u2[:, 16:],
                             wn2[:, :16], wn2[:, 16:]], axis=0)


def _final_body(s_ref, cell_ref, rm2_ref, rv2_ref, g2_ref, be2_ref,
                outt_ref):
    pid = pl.program_id(0)
    sv = s_ref[...]
    agg = jnp.concatenate([sv[0], sv[1]], axis=1)
    scale = g2_ref[...] * lax.rsqrt(rv2_ref[...] + EPS)
    h2 = jnp.maximum((agg - rm2_ref[...]) * scale + be2_ref[...], 0.0)

    rows = pid * BA + lax.broadcasted_iota(jnp.int32, (BA, 1), 0)
    valid = rows < N
    cids = lax.broadcasted_iota(jnp.int32, (1, NUM_GRIDS), 1)
    m = ((cell_ref[...] == cids) & valid).astype(jnp.float32)

    @pl.when(pid == 0)
    def _():
        outt_ref[...] = jnp.zeros((32, NUM_GRIDS), jnp.float32)

    for f in range(32):
        v = jnp.max(m * h2[:, f:f + 1], axis=0)[None, :]
        outt_ref[f:f + 1, :] = jnp.maximum(outt_ref[f:f + 1, :], v)


def _row_spec(w):
    return pl.BlockSpec((BA, w), lambda i: (i, 0))


def _pair_spec(n, w):
    return pl.BlockSpec((n, BA, w), lambda i: (0, i, 0))


def _full_spec(shape):
    return pl.BlockSpec(shape, lambda i: tuple(0 for _ in shape))


def kernel(x, pos, edge_index, W1, b1, rm1, rv1, g1, beta1,
           W2, b2, rm2, rv2, g2, beta2):
    f32 = jnp.float32
    xpad = jnp.pad(x, ((0, NPAD - N), (0, 0)))
    pospad = jnp.pad(pos, ((0, NPAD - N), (0, 0)))
    src0 = jnp.concatenate(
        [edge_index[0], jnp.zeros((EPAD - E,), jnp.int32)]
    ).reshape(16, NBW, 128)
    dst0 = jnp.concatenate(
        [edge_index[1], jnp.full((EPAD - E,), NPAD - 1, jnp.int32)]
    ).reshape(16, NBW, 128)
    srcg = jnp.stack([src0, src0 + NPAD])
    dstg = jnp.stack([dst0 + 2 * NPAD, dst0 + 3 * NPAD])
    z16 = jnp.zeros((RPT, FH), f32)

    t1, wn2, cell = pl.pallas_call(
        _prep_body,
        grid=(NBLK,),
        in_specs=[_row_spec(1), _row_spec(2), _full_spec((3, 16)),
                  _full_spec((2, 32)), _full_spec((1, 16)),
                  _full_spec((1, 32))],
        out_specs=[_pair_spec(4, 16), _row_spec(32), _row_spec(1)],
        out_shape=[jax.ShapeDtypeStruct((4, NPAD, 16), f32),
                   jax.ShapeDtypeStruct((NPAD, 32), f32),
                   jax.ShapeDtypeStruct((NPAD, 1), jnp.int32)],
    )(xpad, pospad, W1, W2[16:18], b1.reshape(1, 16), b2.reshape(1, 32))

    seg = _make_seg_sum()
    (s1,) = seg(t1.reshape(4 * NPAD, FH), srcg, dstg, dst0, z16)

    t2 = pl.pallas_call(
        _mid_body,
        grid=(NBLK,),
        in_specs=[_pair_spec(2, 16), _row_spec(32), _full_spec((16, 32)),
                  _full_spec((1, 32))] + [_full_spec((1, 16))] * 4,
        out_specs=_pair_spec(4, 16),
        out_shape=jax.ShapeDtypeStruct((4, NPAD, 16), f32),
    )(s1.reshape(2, NPAD, FH), wn2, W2[:16], b2.reshape(1, 32),
      rm1.reshape(1, 16), rv1.reshape(1, 16),
      g1.reshape(1, 16), beta1.reshape(1, 16))

    (s2,) = seg(t2.reshape(4 * NPAD, FH), srcg, dstg, dst0, z16)

    outt = pl.pallas_call(
        _final_body,
        grid=(NBLK,),
        in_specs=[_pair_spec(2, 16), _row_spec(1)]
                 + [_full_spec((1, 32))] * 4,
        out_specs=_full_spec((32, NUM_GRIDS)),
        out_shape=jax.ShapeDtypeStruct((32, NUM_GRIDS), f32),
    )(s2.reshape(2, NPAD, FH), cell,
      rm2.reshape(1, 32), rv2.reshape(1, 32),
      g2.reshape(1, 32), beta2.reshape(1, 32))

    return outt.T


# speedup vs baseline: 13.4307x; 1.3558x over previous
import jax
import jax.numpy as jnp
from jax import lax
from jax.experimental import pallas as pl
from jax.experimental.pallas import tpu as pltpu
from jax.experimental.pallas import tpu_sc as plsc

N = 50000
E = 800000
CELL_INV = 1.0 / 16.0
GRID_W = 8
NUM_GRIDS = 64
EPS = 1e-5

NPAD = 51200
RPT = NPAD // 16
EPAD = 802816
EBLK = 256
NBW = EPAD // 16 // EBLK
NQ = 4
NBQ = NBW // NQ
BA = 2048
NBLK = NPAD // BA
FH = 16


def _make_seg_sum():
    mesh = plsc.VectorSubcoreMesh(core_axis_name="c", subcore_axis_name="s",
                                  num_cores=1)

    def body(tab_hbm, srcg_hbm, dstg_hbm, dsts_hbm, z_hbm, s_out,
             srcv, dstgv, dstsv, rowsu, rowsw, semu, semw, sems, acc):
        t = lax.axis_index("s")
        pltpu.sync_copy(z_hbm, acc.at[pl.ds(t * RPT, RPT)])
        plsc.subcore_barrier()

        def chunk(q, carry0):
            pltpu.sync_copy(srcg_hbm.at[t, pl.ds(q * NBQ, NBQ)], srcv)
            pltpu.sync_copy(dstg_hbm.at[t, pl.ds(q * NBQ, NBQ)], dstgv)
            pltpu.sync_copy(dsts_hbm.at[t, pl.ds(q * NBQ, NBQ)], dstsv)
            for p in range(3):
                pltpu.async_copy(tab_hbm.at[srcv.at[p]], rowsu.at[p], semu)
                pltpu.async_copy(tab_hbm.at[dstgv.at[p]], rowsw.at[p], semw)

            def step(j, carry):
                @pl.when(j >= 1)
                def _():
                    pltpu.make_async_copy(
                        rowsu.at[0], acc.at[dstsv.at[0]], sems).wait()
                    pltpu.make_async_copy(
                        rowsw.at[0], acc.at[dstsv.at[0]], sems).wait()

                pltpu.make_async_copy(
                    tab_hbm.at[srcv.at[j]], rowsu.at[j % 4], semu).wait()
                pltpu.make_async_copy(
                    tab_hbm.at[dstgv.at[j]], rowsw.at[j % 4], semw).wait()

                pltpu.async_copy(rowsu.at[j % 4], acc.at[dstsv.at[j]],
                                 sems, add=True)
                pltpu.async_copy(rowsw.at[j % 4], acc.at[dstsv.at[j]],
                                 sems, add=True)

                @pl.when(j + 3 < NBQ)
                def _():
                    pltpu.async_copy(
                        tab_hbm.at[srcv.at[j + 3]], rowsu.at[(j + 3) % 4],
                        semu)
                    pltpu.async_copy(
                        tab_hbm.at[dstgv.at[j + 3]], rowsw.at[(j + 3) % 4],
                        semw)

                return carry

            lax.fori_loop(0, NBQ, step, 0)
            pltpu.make_async_copy(rowsu.at[0], acc.at[dstsv.at[0]],
                                  sems).wait()
            pltpu.make_async_copy(rowsw.at[0], acc.at[dstsv.at[0]],
                                  sems).wait()
            return carry0

        lax.fori_loop(0, NQ, chunk, 0)
        plsc.subcore_barrier()
        pltpu.sync_copy(acc.at[pl.ds(t * RPT, RPT)],
                        s_out.at[pl.ds(t * RPT, RPT)])

    return pl.kernel(
        body,
        out_type=(jax.ShapeDtypeStruct((NPAD, FH), jnp.float32),),
        scratch_types=[
            pltpu.VMEM((NBQ, EBLK), jnp.int32),
            pltpu.VMEM((NBQ, EBLK), jnp.int32),
            pltpu.VMEM((NBQ, EBLK), jnp.int32),
            pltpu.VMEM((4, EBLK, FH), jnp.float32),
            pltpu.VMEM((4, EBLK, FH), jnp.float32),
            pltpu.SemaphoreType.DMA,
            pltpu.SemaphoreType.DMA,
            pltpu.SemaphoreType.DMA,
            pltpu.VMEM_SHARED((NPAD, FH), jnp.float32),
        ],
        mesh=mesh,
        compiler_params=pltpu.CompilerParams(use_tc_tiling_on_sc=False))


def _prep_body(x_ref, pos_ref, w1_ref, w2p_ref, b1_ref, b2_ref,
               t1_ref, wn2_ref, cell_ref):
    xv = x_ref[...]
    pv = pos_ref[...]
    w1 = w1_ref[...]
    w2p = w2p_ref[...]
    px = pv[:, 0:1]
    py = pv[:, 1:2]
    p1 = px * w1[1:2, :] + py * w1[2:3, :]
    u1 = xv * w1[0:1, :] + p1
    wn1 = b1_ref[...] - p1
    t1_ref[...] = jnp.stack([u1, wn1], axis=0)
    wn2_ref[...] = b2_ref[...] - (px * w2p[0:1, :] + py * w2p[1:2, :])
    ix = jnp.clip(jnp.floor(px * CELL_INV), 0, GRID_W - 1).astype(jnp.int32)
    iy = jnp.clip(jnp.floor(py * CELL_INV), 0, GRID_W - 1).astype(jnp.int32)
    cell_ref[...] = iy * GRID_W + ix


def _mid_body(s_ref, wn2_ref, w2h_ref, b2_ref,
              rm1_ref, rv1_ref, g1_ref, be1_ref, t2a_ref, t2b_ref):
    s = s_ref[...]
    scale = g1_ref[...] * lax.rsqrt(rv1_ref[...] + EPS)
    h1 = jnp.maximum((s - rm1_ref[...]) * scale + be1_ref[...], 0.0)
    wn2 = wn2_ref[...]
    u2 = jnp.dot(h1, w2h_ref[...],
                 preferred_element_type=jnp.float32) + (b2_ref[...] - wn2)
    t2a_ref[...] = jnp.stack([u2[:, :16], wn2[:, :16]], axis=0)
    t2b_ref[...] = jnp.stack([u2[:, 16:], wn2[:, 16:]], axis=0)


def _final_body(sa_ref, sb_ref, cell_ref, rm2_ref, rv2_ref, g2_ref, be2_ref,
                outt_ref):
    pid = pl.program_id(0)
    agg = jnp.concatenate([sa_ref[...], sb_ref[...]], axis=1)
    scale = g2_ref[...] * lax.rsqrt(rv2_ref[...] + EPS)
    h2 = jnp.maximum((agg - rm2_ref[...]) * scale + be2_ref[...], 0.0)

    rows = pid * BA + lax.broadcasted_iota(jnp.int32, (BA, 1), 0)
    valid = rows < N
    cids = lax.broadcasted_iota(jnp.int32, (1, NUM_GRIDS), 1)
    m = ((cell_ref[...] == cids) & valid).astype(jnp.float32)

    @pl.when(pid == 0)
    def _():
        outt_ref[...] = jnp.zeros((32, NUM_GRIDS), jnp.float32)

    for f in range(32):
        v = jnp.max(m * h2[:, f:f + 1], axis=0)[None, :]
        outt_ref[f:f + 1, :] = jnp.maximum(outt_ref[f:f + 1, :], v)


def _row_spec(w):
    return pl.BlockSpec((BA, w), lambda i: (i, 0))


def _pair_spec(n, w):
    return pl.BlockSpec((n, BA, w), lambda i: (0, i, 0))


def _full_spec(shape):
    return pl.BlockSpec(shape, lambda i: tuple(0 for _ in shape))


def kernel(x, pos, edge_index, W1, b1, rm1, rv1, g1, beta1,
           W2, b2, rm2, rv2, g2, beta2):
    f32 = jnp.float32
    xpad = jnp.pad(x, ((0, NPAD - N), (0, 0)))
    pospad = jnp.pad(pos, ((0, NPAD - N), (0, 0)))
    srcg = jnp.concatenate(
        [edge_index[0], jnp.zeros((EPAD - E,), jnp.int32)]
    ).reshape(16, NBW, EBLK)
    dsts = jnp.concatenate(
        [edge_index[1], jnp.full((EPAD - E,), NPAD - 1, jnp.int32)]
    ).reshape(16, NBW, EBLK)
    dstg = dsts + NPAD
    z16 = jnp.zeros((RPT, FH), f32)

    t1, wn2, cell = pl.pallas_call(
        _prep_body,
        grid=(NBLK,),
        in_specs=[_row_spec(1), _row_spec(2), _full_spec((3, 16)),
                  _full_spec((2, 32)), _full_spec((1, 16)),
                  _full_spec((1, 32))],
        out_specs=[_pair_spec(2, 16), _row_spec(32), _row_spec(1)],
        out_shape=[jax.ShapeDtypeStruct((2, NPAD, 16), f32),
                   jax.ShapeDtypeStruct((NPAD, 32), f32),
                   jax.ShapeDtypeStruct((NPAD, 1), jnp.int32)],
    )(xpad, pospad, W1, W2[16:18], b1.reshape(1, 16), b2.reshape(1, 32))

    seg = _make_seg_sum()
    (s1,) = seg(t1.reshape(2 * NPAD, FH), srcg, dstg, dsts, z16)

    t2a, t2b = pl.pallas_call(
        _mid_body,
        grid=(NBLK,),
        in_specs=[_row_spec(16), _row_spec(32), _full_spec((16, 32)),
                  _full_spec((1, 32))] + [_full_spec((1, 16))] * 4,
        out_specs=[_pair_spec(2, 16), _pair_spec(2, 16)],
        out_shape=[jax.ShapeDtypeStruct((2, NPAD, 16), f32),
                   jax.ShapeDtypeStruct((2, NPAD, 16), f32)],
    )(s1, wn2, W2[:16], b2.reshape(1, 32),
      rm1.reshape(1, 16), rv1.reshape(1, 16),
      g1.reshape(1, 16), beta1.reshape(1, 16))

    (s2a,) = seg(t2a.reshape(2 * NPAD, FH), srcg, dstg, dsts, z16)
    (s2b,) = seg(t2b.reshape(2 * NPAD, FH), srcg, dstg, dsts, z16)

    outt = pl.pallas_call(
        _final_body,
        grid=(NBLK,),
        in_specs=[_row_spec(16), _row_spec(16), _row_spec(1)]
                 + [_full_spec((1, 32))] * 4,
        out_specs=_full_spec((32, NUM_GRIDS)),
        out_shape=jax.ShapeDtypeStruct((32, NUM_GRIDS), f32),
    )(s2a, s2b, cell,
      rm2.reshape(1, 32), rv2.reshape(1, 32),
      g2.reshape(1, 32), beta2.reshape(1, 32))

    return outt.T
